# Initial kernel scaffold; baseline (speedup 1.0000x reference)
import jax
import jax.numpy as jnp
from jax.experimental import pallas as pl


def kernel(x, w1, b1, w2, b2, w3, b3):
    raise NotImplementedError("write your pallas kernel here")


# speedup vs baseline: 1.8594x; 1.8594x over previous
import functools

import jax
import jax.numpy as jnp
from jax.experimental import pallas as pl
from jax.experimental.pallas import tpu as pltpu


def _whl_body(x_ref, a1_ref, b1_ref, a2_ref, b2_ref, w3_ref, b3_ref, out_ref,
              *, window_size, n_w, C, C2, C4, L):
    W = C * L
    x = x_ref[...].astype(jnp.float32)

    m = x
    span = 1
    while span * 2 <= window_size:
        m = jnp.maximum(m, pltpu.roll(m, W - span, 1))
        span *= 2
    if span < window_size:
        s = window_size - span
        m = jnp.maximum(m, pltpu.roll(m, W - s, 1))

    mm = jnp.concatenate([m, m[:, :C2 * L]], axis=1)
    h = mm[:, 0:C2 * L] * a1_ref[0:1, :]
    for k in range(1, C):
        h = h + mm[:, k * L:k * L + C2 * L] * a1_ref[k:k + 1, :]
    h = h + b1_ref[0:1, :]
    h = h * jax.nn.sigmoid(h)

    hh = jnp.concatenate([h, h[:, :C4 * L]], axis=1)
    g = hh[:, 0:C4 * L] * a2_ref[0:1, :]
    for k in range(1, C2):
        g = g + hh[:, k * L:k * L + C4 * L] * a2_ref[k:k + 1, :]
    g = g + b2_ref[0:1, :]
    g = g * jax.nn.sigmoid(g)

    logits = g[:, 0:L] * w3_ref[0]
    for c in range(1, C4):
        logits = logits + g[:, c * L:(c + 1) * L] * w3_ref[c]
    logits = logits + b3_ref[0]

    col = jax.lax.broadcasted_iota(jnp.int32, logits.shape, 1)
    neg = jnp.finfo(jnp.float32).min
    out_ref[...] = jnp.max(jnp.where(col < n_w, logits, neg),
                           axis=1, keepdims=True)


def _segment_patterns(w1, w2, b1, b2, C, C2, C4, L):
    f32 = jnp.float32
    k1 = jnp.arange(C)[:, None]
    s1 = jnp.arange(C2)[None, :]
    p1 = jnp.asarray(w1, f32)[s1, (s1 + k1) % C]
    a1 = jnp.repeat(p1, L, axis=1)
    k2 = jnp.arange(C2)[:, None]
    s2 = jnp.arange(C4)[None, :]
    p2 = jnp.asarray(w2, f32)[s2, (s2 + k2) % C2]
    a2 = jnp.repeat(p2, L, axis=1)
    b1p = jnp.repeat(jnp.asarray(b1, f32), L)[None, :]
    b2p = jnp.repeat(jnp.asarray(b2, f32), L)[None, :]
    return a1, a2, b1p, b2p


def kernel(x, w1, b1, w2, b2, w3, b3):
    window_size = 16
    B, C, L = x.shape
    C2, C4 = w1.shape[0], w2.shape[0]
    n_w = L - window_size + 1
    W = C * L

    xf = x.reshape(B, W)
    a1, a2, b1p, b2p = _segment_patterns(w1, w2, b1, b2, C, C2, C4, L)

    itemsize = int(jnp.dtype(x.dtype).itemsize)
    Bt = int(max(8, min(B, (2 << 20) // max(1, W * itemsize))))
    n_blocks = pl.cdiv(B, Bt)
    Bpad = n_blocks * Bt
    if Bpad != B:
        xf = jnp.pad(xf, ((0, Bpad - B), (0, 0)))

    f32 = jnp.float32
    smem = pl.BlockSpec(memory_space=pltpu.MemorySpace.SMEM)
    body = functools.partial(_whl_body, window_size=window_size, n_w=n_w,
                             C=C, C2=C2, C4=C4, L=L)

    out = pl.pallas_call(
        body,
        out_shape=jax.ShapeDtypeStruct((Bpad, 1), f32),
        grid=(n_blocks,),
        in_specs=[
            pl.BlockSpec((Bt, W), lambda b: (b, 0)),
            pl.BlockSpec((C, C2 * L), lambda b: (0, 0)),
            pl.BlockSpec((1, C2 * L), lambda b: (0, 0)),
            pl.BlockSpec((C2, C4 * L), lambda b: (0, 0)),
            pl.BlockSpec((1, C4 * L), lambda b: (0, 0)),
            smem, smem,
        ],
        out_specs=pl.BlockSpec((Bt, 1), lambda b: (b, 0)),
        compiler_params=pltpu.CompilerParams(
            dimension_semantics=("parallel",),
            vmem_limit_bytes=64 * 1024 * 1024),
        cost_estimate=pl.CostEstimate(
            flops=2 * B * n_w * (C * C2 + C2 * C4 + C4),
            transcendentals=B * n_w * (C2 + C4),
            bytes_accessed=B * W * itemsize + B * 4),
    )(xf, a1, b1p, a2, b2p, jnp.asarray(w3, f32), jnp.asarray(b3, f32))

    return out[:B]


# speedup vs baseline: 2.1519x; 1.1573x over previous
import functools

import jax
import jax.numpy as jnp
from jax.experimental import pallas as pl
from jax.experimental.pallas import tpu as pltpu

_G = 16


def _sliding_max(m, window_size, L):
    span = 1
    while span * 2 <= window_size:
        m = jnp.maximum(m, pltpu.roll(m, L - span, 1))
        span *= 2
    if span < window_size:
        s = window_size - span
        m = jnp.maximum(m, pltpu.roll(m, L - s, 1))
    return m


def _whl_body(x_ref, a1_ref, b1_ref, a2_ref, b2_ref, a3_ref, b3_ref, out_ref,
              *, window_size, n_w, C, C2, C4, L, n_chunks):
    R = _G * C
    f32 = jnp.float32
    neg = jnp.finfo(f32).min
    a1 = a1_ref[...]
    a2 = a2_ref[...]
    a3 = a3_ref[...]
    b1 = b1_ref[...]
    b2 = b2_ref[...]
    for k in range(n_chunks):
        xc = x_ref[k * R:(k + 1) * R, :].astype(f32)
        m = _sliding_max(xc, window_size, L)
        h = jnp.dot(a1, m, preferred_element_type=f32) + b1
        h = h * jax.nn.sigmoid(h)
        g = jnp.dot(a2, h, preferred_element_type=f32) + b2
        g = g * jax.nn.sigmoid(g)
        logits = jnp.dot(a3, g, preferred_element_type=f32) + b3_ref[0]
        col = jax.lax.broadcasted_iota(jnp.int32, logits.shape, 1)
        res = jnp.max(jnp.where(col < n_w, logits, neg),
                      axis=1, keepdims=True)
        out_ref[k * _G:(k + 1) * _G, :] = res


def kernel(x, w1, b1, w2, b2, w3, b3):
    window_size = 16
    B, C, L = x.shape
    C2, C4 = w1.shape[0], w2.shape[0]
    n_w = L - window_size + 1

    itemsize = int(jnp.dtype(x.dtype).itemsize)
    Bt = int(max(_G, min(B, (2 << 20) // max(1, C * L * itemsize))))
    Bt -= Bt % _G
    n_blocks = pl.cdiv(B, Bt)
    Bpad = n_blocks * Bt
    x_in = x
    if Bpad != B:
        x_in = jnp.pad(x, ((0, Bpad - B), (0, 0), (0, 0)))
    xf = x_in.reshape(Bpad * C, L)
    n_chunks = Bt // _G

    f32 = jnp.float32
    eye = jnp.eye(_G, dtype=f32)
    a1 = jnp.kron(eye, jnp.asarray(w1, f32))
    a2 = jnp.kron(eye, jnp.asarray(w2, f32))
    a3 = jnp.kron(eye, jnp.asarray(w3, f32)[None, :])
    b1t = jnp.tile(jnp.asarray(b1, f32), _G)[:, None]
    b2t = jnp.tile(jnp.asarray(b2, f32), _G)[:, None]

    smem = pl.BlockSpec(memory_space=pltpu.MemorySpace.SMEM)
    body = functools.partial(_whl_body, window_size=window_size, n_w=n_w,
                             C=C, C2=C2, C4=C4, L=L, n_chunks=n_chunks)

    out = pl.pallas_call(
        body,
        out_shape=jax.ShapeDtypeStruct((Bpad, 1), f32),
        grid=(n_blocks,),
        in_specs=[
            pl.BlockSpec((Bt * C, L), lambda b: (b, 0)),
            pl.BlockSpec((_G * C2, _G * C), lambda b: (0, 0)),
            pl.BlockSpec((_G * C2, 1), lambda b: (0, 0)),
            pl.BlockSpec((_G * C4, _G * C2), lambda b: (0, 0)),
            pl.BlockSpec((_G * C4, 1), lambda b: (0, 0)),
            pl.BlockSpec((_G, _G * C4), lambda b: (0, 0)),
            smem,
        ],
        out_specs=pl.BlockSpec((Bt, 1), lambda b: (b, 0)),
        compiler_params=pltpu.CompilerParams(
            dimension_semantics=("parallel",),
            vmem_limit_bytes=64 * 1024 * 1024),
        cost_estimate=pl.CostEstimate(
            flops=2 * B * n_w * (C * C2 + C2 * C4 + C4),
            transcendentals=B * n_w * (C2 + C4),
            bytes_accessed=B * C * L * itemsize + B * 4),
    )(xf, a1, b1t, a2, b2t, a3, jnp.asarray(b3, f32))

    return out[:B]


# speedup vs baseline: 2.6314x; 1.2228x over previous
import functools

import jax
import jax.numpy as jnp
from jax.experimental import pallas as pl
from jax.experimental.pallas import tpu as pltpu

_G = 16
_S = 4


def _sliding_max(m, window_size, L):
    span = 1
    while span * 2 <= window_size:
        m = jnp.maximum(m, pltpu.roll(m, L - span, 1))
        span *= 2
    if span < window_size:
        s = window_size - span
        m = jnp.maximum(m, pltpu.roll(m, L - s, 1))
    return m


def _whl_body(x_ref, a1_ref, b1_ref, a2_ref, b2_ref, a3_ref, b3_ref, out_ref,
              *, window_size, n_w, C, C2, C4, L, n_chunks):
    R = _G * C
    f32 = jnp.float32
    neg = jnp.finfo(f32).min
    a1 = a1_ref[...]
    a2 = a2_ref[...]
    a3 = a3_ref[...]
    b1 = b1_ref[...]
    b2 = b2_ref[...]
    for k0 in range(0, n_chunks, _S):
        ks = range(k0, min(k0 + _S, n_chunks))
        ms = [_sliding_max(x_ref[k * R:(k + 1) * R, :].astype(f32),
                           window_size, L) for k in ks]
        hs = [jnp.dot(a1, m, preferred_element_type=f32) + b1 for m in ms]
        hs = [h * jax.nn.sigmoid(h) for h in hs]
        gs = [jnp.dot(a2, h, preferred_element_type=f32) + b2 for h in hs]
        gs = [g * jax.nn.sigmoid(g) for g in gs]
        ls = [jnp.dot(a3, g, preferred_element_type=f32) + b3_ref[0]
              for g in gs]
        for k, logits in zip(ks, ls):
            col = jax.lax.broadcasted_iota(jnp.int32, logits.shape, 1)
            res = jnp.max(jnp.where(col < n_w, logits, neg),
                          axis=1, keepdims=True)
            out_ref[k * _G:(k + 1) * _G, :] = res


def kernel(x, w1, b1, w2, b2, w3, b3):
    window_size = 16
    B, C, L = x.shape
    C2, C4 = w1.shape[0], w2.shape[0]
    n_w = L - window_size + 1

    itemsize = int(jnp.dtype(x.dtype).itemsize)
    Bt = int(max(_G, min(B, (2 << 20) // max(1, C * L * itemsize))))
    Bt -= Bt % _G
    n_blocks = pl.cdiv(B, Bt)
    Bpad = n_blocks * Bt
    x_in = x
    if Bpad != B:
        x_in = jnp.pad(x, ((0, Bpad - B), (0, 0), (0, 0)))
    xf = x_in.reshape(Bpad * C, L)
    n_chunks = Bt // _G

    f32 = jnp.float32
    eye = jnp.eye(_G, dtype=f32)
    a1 = jnp.kron(eye, jnp.asarray(w1, f32))
    a2 = jnp.kron(eye, jnp.asarray(w2, f32))
    a3 = jnp.kron(eye, jnp.asarray(w3, f32)[None, :])
    b1t = jnp.tile(jnp.asarray(b1, f32), _G)[:, None]
    b2t = jnp.tile(jnp.asarray(b2, f32), _G)[:, None]

    smem = pl.BlockSpec(memory_space=pltpu.MemorySpace.SMEM)
    body = functools.partial(_whl_body, window_size=window_size, n_w=n_w,
                             C=C, C2=C2, C4=C4, L=L, n_chunks=n_chunks)

    out = pl.pallas_call(
        body,
        out_shape=jax.ShapeDtypeStruct((Bpad, 1), f32),
        grid=(n_blocks,),
        in_specs=[
            pl.BlockSpec((Bt * C, L), lambda b: (b, 0)),
            pl.BlockSpec((_G * C2, _G * C), lambda b: (0, 0)),
            pl.BlockSpec((_G * C2, 1), lambda b: (0, 0)),
            pl.BlockSpec((_G * C4, _G * C2), lambda b: (0, 0)),
            pl.BlockSpec((_G * C4, 1), lambda b: (0, 0)),
            pl.BlockSpec((_G, _G * C4), lambda b: (0, 0)),
            smem,
        ],
        out_specs=pl.BlockSpec((Bt, 1), lambda b: (b, 0)),
        compiler_params=pltpu.CompilerParams(
            dimension_semantics=("arbitrary",),
            vmem_limit_bytes=64 * 1024 * 1024),
        cost_estimate=pl.CostEstimate(
            flops=2 * B * n_w * (C * C2 + C2 * C4 + C4),
            transcendentals=B * n_w * (C2 + C4),
            bytes_accessed=B * C * L * itemsize + B * 4),
    )(xf, a1, b1t, a2, b2t, a3, jnp.asarray(b3, f32))

    return out[:B]


# speedup vs baseline: 4.1629x; 1.5820x over previous
import functools

import jax
import jax.numpy as jnp
from jax.experimental import pallas as pl
from jax.experimental.pallas import tpu as pltpu

_G = 16
_S = 4


def _sliding_max(m, window_size, L):
    span = 1
    while span * 2 <= window_size:
        m = jnp.maximum(m, pltpu.roll(m, L - span, 1))
        span *= 2
    if span < window_size:
        s = window_size - span
        m = jnp.maximum(m, pltpu.roll(m, L - s, 1))
    return m


def _whl_body(x_ref, a1_ref, b1_ref, a2_ref, b2_ref, a3_ref, b3_ref, out_ref,
              *, window_size, n_w, C, C2, C4, L, n_chunks):
    R = _G * C
    f32 = jnp.float32
    neg = jnp.finfo(f32).min
    a1 = a1_ref[...]
    a2 = a2_ref[...]
    a3 = a3_ref[...]
    b1 = b1_ref[...]
    b2 = b2_ref[...]
    for k0 in range(0, n_chunks, _S):
        ks = range(k0, min(k0 + _S, n_chunks))
        ms = [_sliding_max(x_ref[k * R:(k + 1) * R, :].astype(jnp.bfloat16),
                           window_size, L) for k in ks]
        hs = [jnp.dot(a1, m, preferred_element_type=f32) + b1 for m in ms]
        hs = [h * jax.nn.sigmoid(h) for h in hs]
        gs = [jnp.dot(a2, h, preferred_element_type=f32) + b2 for h in hs]
        gs = [g * jax.nn.sigmoid(g) for g in gs]
        ls = [jnp.dot(a3, g, preferred_element_type=f32) + b3_ref[0]
              for g in gs]
        for k, logits in zip(ks, ls):
            col = jax.lax.broadcasted_iota(jnp.int32, logits.shape, 1)
            res = jnp.max(jnp.where(col < n_w, logits, neg),
                          axis=1, keepdims=True)
            out_ref[k * _G:(k + 1) * _G, :] = res


def kernel(x, w1, b1, w2, b2, w3, b3):
    window_size = 16
    B, C, L = x.shape
    C2, C4 = w1.shape[0], w2.shape[0]
    n_w = L - window_size + 1

    itemsize = int(jnp.dtype(x.dtype).itemsize)
    Bt = int(max(_G, min(B, (2 << 20) // max(1, C * L * itemsize))))
    Bt -= Bt % _G
    n_blocks = pl.cdiv(B, Bt)
    Bpad = n_blocks * Bt
    x_in = x
    if Bpad != B:
        x_in = jnp.pad(x, ((0, Bpad - B), (0, 0), (0, 0)))
    xf = x_in.reshape(Bpad * C, L)
    n_chunks = Bt // _G

    f32 = jnp.float32
    eye = jnp.eye(_G, dtype=f32)
    a1 = jnp.kron(eye, jnp.asarray(w1, f32)).astype(jnp.bfloat16)
    a2 = jnp.kron(eye, jnp.asarray(w2, f32))
    a3 = jnp.kron(eye, jnp.asarray(w3, f32)[None, :])
    b1t = jnp.tile(jnp.asarray(b1, f32), _G)[:, None]
    b2t = jnp.tile(jnp.asarray(b2, f32), _G)[:, None]

    smem = pl.BlockSpec(memory_space=pltpu.MemorySpace.SMEM)
    body = functools.partial(_whl_body, window_size=window_size, n_w=n_w,
                             C=C, C2=C2, C4=C4, L=L, n_chunks=n_chunks)

    out = pl.pallas_call(
        body,
        out_shape=jax.ShapeDtypeStruct((Bpad, 1), f32),
        grid=(n_blocks,),
        in_specs=[
            pl.BlockSpec((Bt * C, L), lambda b: (b, 0)),
            pl.BlockSpec((_G * C2, _G * C), lambda b: (0, 0)),
            pl.BlockSpec((_G * C2, 1), lambda b: (0, 0)),
            pl.BlockSpec((_G * C4, _G * C2), lambda b: (0, 0)),
            pl.BlockSpec((_G * C4, 1), lambda b: (0, 0)),
            pl.BlockSpec((_G, _G * C4), lambda b: (0, 0)),
            smem,
        ],
        out_specs=pl.BlockSpec((Bt, 1), lambda b: (b, 0)),
        compiler_params=pltpu.CompilerParams(
            dimension_semantics=("arbitrary",),
            vmem_limit_bytes=64 * 1024 * 1024),
        cost_estimate=pl.CostEstimate(
            flops=2 * B * n_w * (C * C2 + C2 * C4 + C4),
            transcendentals=B * n_w * (C2 + C4),
            bytes_accessed=B * C * L * itemsize + B * 4),
    )(xf, a1, b1t, a2, b2t, a3, jnp.asarray(b3, f32))

    return out[:B]


# speedup vs baseline: 4.6001x; 1.1050x over previous
import functools

import jax
import jax.numpy as jnp
from jax.experimental import pallas as pl
from jax.experimental.pallas import tpu as pltpu

_G = 16
_S = 4


def _sliding_max(m, window_size, L):
    span = 1
    while span * 2 <= window_size:
        m = jnp.maximum(m, pltpu.roll(m, L - span, 1))
        span *= 2
    if span < window_size:
        s = window_size - span
        m = jnp.maximum(m, pltpu.roll(m, L - s, 1))
    return m


def _whl_body(x_ref, a1_ref, b1_ref, a2_ref, b2_ref, a3_ref, b3_ref, out_ref,
              *, window_size, n_w, C, C2, C4, L, n_chunks):
    R = _G * C
    f32 = jnp.float32
    neg = jnp.finfo(f32).min
    a1 = a1_ref[...]
    a2 = a2_ref[...]
    a3 = a3_ref[...]
    b1 = b1_ref[...]
    b2 = b2_ref[...]
    for k0 in range(0, n_chunks, _S):
        ks = range(k0, min(k0 + _S, n_chunks))
        ms = [_sliding_max(x_ref[k * R:(k + 1) * R, :].astype(jnp.bfloat16),
                           window_size, L) for k in ks]
        hs = [jnp.dot(a1, m, preferred_element_type=f32) + b1 for m in ms]
        hs = [h * jax.nn.sigmoid(h) for h in hs]
        gs = [jnp.dot(a2, h, preferred_element_type=f32) + b2 for h in hs]
        gs = [g * jax.nn.sigmoid(g) for g in gs]
        ls = [jnp.dot(a3, g, preferred_element_type=f32) + b3_ref[0]
              for g in gs]
        for k, logits in zip(ks, ls):
            col = jax.lax.broadcasted_iota(jnp.int32, logits.shape, 1)
            res = jnp.max(jnp.where(col < n_w, logits, neg),
                          axis=1, keepdims=True)
            out_ref[k * _G:(k + 1) * _G, :] = res


def kernel(x, w1, b1, w2, b2, w3, b3):
    window_size = 16
    B, C, L = x.shape
    C2, C4 = w1.shape[0], w2.shape[0]
    n_w = L - window_size + 1

    itemsize = int(jnp.dtype(x.dtype).itemsize)
    Bt = int(max(_G, min(B, (4 << 20) // max(1, C * L * itemsize))))
    Bt -= Bt % _G
    n_blocks = pl.cdiv(B, Bt)
    Bpad = n_blocks * Bt
    x_in = x
    if Bpad != B:
        x_in = jnp.pad(x, ((0, Bpad - B), (0, 0), (0, 0)))
    xf = x_in.reshape(Bpad * C, L)
    n_chunks = Bt // _G

    f32 = jnp.float32
    eye = jnp.eye(_G, dtype=f32)
    a1 = jnp.kron(eye, jnp.asarray(w1, f32)).astype(jnp.bfloat16)
    a2 = jnp.kron(eye, jnp.asarray(w2, f32))
    a3 = jnp.kron(eye, jnp.asarray(w3, f32)[None, :])
    b1t = jnp.tile(jnp.asarray(b1, f32), _G)[:, None]
    b2t = jnp.tile(jnp.asarray(b2, f32), _G)[:, None]

    smem = pl.BlockSpec(memory_space=pltpu.MemorySpace.SMEM)
    body = functools.partial(_whl_body, window_size=window_size, n_w=n_w,
                             C=C, C2=C2, C4=C4, L=L, n_chunks=n_chunks)

    out = pl.pallas_call(
        body,
        out_shape=jax.ShapeDtypeStruct((Bpad, 1), f32),
        grid=(n_blocks,),
        in_specs=[
            pl.BlockSpec((Bt * C, L), lambda b: (b, 0)),
            pl.BlockSpec((_G * C2, _G * C), lambda b: (0, 0)),
            pl.BlockSpec((_G * C2, 1), lambda b: (0, 0)),
            pl.BlockSpec((_G * C4, _G * C2), lambda b: (0, 0)),
            pl.BlockSpec((_G * C4, 1), lambda b: (0, 0)),
            pl.BlockSpec((_G, _G * C4), lambda b: (0, 0)),
            smem,
        ],
        out_specs=pl.BlockSpec((Bt, 1), lambda b: (b, 0)),
        compiler_params=pltpu.CompilerParams(
            dimension_semantics=("arbitrary",),
            vmem_limit_bytes=64 * 1024 * 1024),
        cost_estimate=pl.CostEstimate(
            flops=2 * B * n_w * (C * C2 + C2 * C4 + C4),
            transcendentals=B * n_w * (C2 + C4),
            bytes_accessed=B * C * L * itemsize + B * 4),
    )(xf, a1, b1t, a2, b2t, a3, jnp.asarray(b3, f32))

    return out[:B]


# speedup vs baseline: 4.7732x; 1.0376x over previous
import functools

import jax
import jax.numpy as jnp
from jax.experimental import pallas as pl
from jax.experimental.pallas import tpu as pltpu

_G = 16
_S = 4


def _sliding_max(m, window_size, L):
    span = 1
    while span * 2 <= window_size:
        m = jnp.maximum(m, pltpu.roll(m, L - span, 1))
        span *= 2
    if span < window_size:
        s = window_size - span
        m = jnp.maximum(m, pltpu.roll(m, L - s, 1))
    return m


def _whl_body(x_ref, a1_ref, b1_ref, a2_ref, b2_ref, a3_ref, b3_ref, out_ref,
              *, window_size, n_w, C, C2, C4, L, n_chunks):
    R = _G * C
    f32 = jnp.float32
    neg = jnp.finfo(f32).min
    a1 = a1_ref[...]
    a2 = a2_ref[...]
    a3 = a3_ref[...]
    b1 = b1_ref[...]
    b2 = b2_ref[...]
    for k0 in range(0, n_chunks, _S):
        ks = range(k0, min(k0 + _S, n_chunks))
        ms = [_sliding_max(x_ref[k * R:(k + 1) * R, :].astype(jnp.bfloat16),
                           window_size, L) for k in ks]
        hs = [jnp.dot(a1, m, preferred_element_type=f32) + b1 for m in ms]
        hs = [h * jax.nn.sigmoid(h) for h in hs]
        gs = [jnp.dot(a2, h, preferred_element_type=f32) + b2 for h in hs]
        gs = [g * jax.nn.sigmoid(g) for g in gs]
        ls = [jnp.dot(a3, g, preferred_element_type=f32) + b3_ref[0]
              for g in gs]
        for k, logits in zip(ks, ls):
            col = jax.lax.broadcasted_iota(jnp.int32, logits.shape, 1)
            res = jnp.max(jnp.where(col < n_w, logits, neg),
                          axis=1, keepdims=True)
            out_ref[k * _G:(k + 1) * _G, :] = res


def kernel(x, w1, b1, w2, b2, w3, b3):
    window_size = 16
    B, C, L = x.shape
    C2, C4 = w1.shape[0], w2.shape[0]
    n_w = L - window_size + 1

    itemsize = int(jnp.dtype(x.dtype).itemsize)
    Bt = int(max(_G, min(B, (8 << 20) // max(1, C * L * itemsize))))
    Bt -= Bt % _G
    n_blocks = pl.cdiv(B, Bt)
    Bpad = n_blocks * Bt
    x_in = x
    if Bpad != B:
        x_in = jnp.pad(x, ((0, Bpad - B), (0, 0), (0, 0)))
    xf = x_in.reshape(Bpad * C, L)
    n_chunks = Bt // _G

    f32 = jnp.float32
    eye = jnp.eye(_G, dtype=f32)
    a1 = jnp.kron(eye, jnp.asarray(w1, f32)).astype(jnp.bfloat16)
    a2 = jnp.kron(eye, jnp.asarray(w2, f32))
    a3 = jnp.kron(eye, jnp.asarray(w3, f32)[None, :])
    b1t = jnp.tile(jnp.asarray(b1, f32), _G)[:, None]
    b2t = jnp.tile(jnp.asarray(b2, f32), _G)[:, None]

    smem = pl.BlockSpec(memory_space=pltpu.MemorySpace.SMEM)
    body = functools.partial(_whl_body, window_size=window_size, n_w=n_w,
                             C=C, C2=C2, C4=C4, L=L, n_chunks=n_chunks)

    out = pl.pallas_call(
        body,
        out_shape=jax.ShapeDtypeStruct((Bpad, 1), f32),
        grid=(n_blocks,),
        in_specs=[
            pl.BlockSpec((Bt * C, L), lambda b: (b, 0)),
            pl.BlockSpec((_G * C2, _G * C), lambda b: (0, 0)),
            pl.BlockSpec((_G * C2, 1), lambda b: (0, 0)),
            pl.BlockSpec((_G * C4, _G * C2), lambda b: (0, 0)),
            pl.BlockSpec((_G * C4, 1), lambda b: (0, 0)),
            pl.BlockSpec((_G, _G * C4), lambda b: (0, 0)),
            smem,
        ],
        out_specs=pl.BlockSpec((Bt, 1), lambda b: (b, 0)),
        compiler_params=pltpu.CompilerParams(
            dimension_semantics=("arbitrary",),
            vmem_limit_bytes=64 * 1024 * 1024),
        cost_estimate=pl.CostEstimate(
            flops=2 * B * n_w * (C * C2 + C2 * C4 + C4),
            transcendentals=B * n_w * (C2 + C4),
            bytes_accessed=B * C * L * itemsize + B * 4),
    )(xf, a1, b1t, a2, b2t, a3, jnp.asarray(b3, f32))

    return out[:B]
